# Initial kernel scaffold; baseline (speedup 1.0000x reference)
import jax
import jax.numpy as jnp
from jax.experimental import pallas as pl


def kernel(x, edge_index, W1, a_src1, a_dst1, W2, a_src2, a_dst2, W3, a_src3, a_dst3):
    raise NotImplementedError("write your pallas kernel here")


# speedup vs baseline: 17.3047x; 17.3047x over previous
import functools

import jax
import jax.numpy as jnp
from jax import lax
from jax.experimental import pallas as pl
from jax.experimental.pallas import tpu as pltpu
from jax.experimental.pallas import tpu_sc as plsc

_N = 10000
_E = 320000
_NPAD = 10240
_SENT = 10000
_NC, _NS = 2, 16
_NW = _NC * _NS
_CHUNK = 128
_RPT = _NPAD // _NS
_NEG = -1e30

_NCH16 = -(-_E // (_NS * _CHUNK))
_EPT16 = _NCH16 * _CHUNK
_EPAD16 = _NS * _EPT16
_NCH32 = -(-_E // (_NW * _CHUNK))
_EPT32 = _NCH32 * _CHUNK
_EPAD32 = _NW * _EPT32


def _logits(h, a_s, a_d, es_ref, ed_ref, mh_ref):
    es = jnp.sum(h * a_s, axis=1, keepdims=True)
    ed = jnp.sum(h * a_d, axis=1, keepdims=True)
    es_ref[0:_N, :] = es
    es_ref[_N:_NPAD, :] = jnp.full((_NPAD - _N, 1), _NEG, jnp.float32)
    ed_ref[0:_N, :] = ed
    ed_ref[_N:_NPAD, :] = jnp.full((_NPAD - _N, 1), _NEG, jnp.float32)
    mh_ref[0:_N, :] = jnp.maximum(jnp.max(es) + ed, 0.0)
    mh_ref[_N:_NPAD, :] = jnp.zeros((_NPAD - _N, 1), jnp.float32)


def _pack_half(h_half, ones_col, hp_ref):
    fh = h_half.shape[1]
    fp = hp_ref.shape[1]
    hp_ref[0:_N, 0:fh] = h_half
    colid = lax.broadcasted_iota(jnp.int32, (_N, fp - fh), 1)
    fill = jnp.where(colid == 0, 1.0, 0.0) if ones_col else jnp.zeros((_N, fp - fh))
    hp_ref[0:_N, fh:fp] = fill.astype(jnp.float32)
    hp_ref[_N:_NPAD, :] = jnp.zeros((_NPAD - _N, fp), jnp.float32)


def _tc_split_body(x_and_w_to_h, sp_ref, w_ref, as_ref, ad_ref,
                   hp0_ref, hp1_ref, es_ref, ed_ref, mh_ref):
    h = x_and_w_to_h(sp_ref, w_ref)
    _pack_half(h[:, 0:64], True, hp0_ref)
    _pack_half(h[:, 64:128], False, hp1_ref)
    _logits(h, as_ref[...], ad_ref[...], es_ref, ed_ref, mh_ref)


def _h_first(x_ref, w_ref):
    return jnp.dot(x_ref[...], w_ref[...], preferred_element_type=jnp.float32)


def _h_mid_split(sp_ref, w_ref):
    s0 = sp_ref[0]
    s1 = sp_ref[1]
    d = s0[0:_N, 64:65]
    dsafe = jnp.where(d > 0.0, d, 1.0)
    hin = jnp.concatenate([s0[0:_N, 0:64], s1[0:_N, 0:64]], axis=1) / dsafe
    return jnp.dot(hin, w_ref[...], preferred_element_type=jnp.float32)


def _tc_mid48_body(sp_ref, w_ref, as_ref, ad_ref,
                   hp_ref, es_ref, ed_ref, mh_ref):
    h = _h_mid_split(sp_ref, w_ref)
    _pack_half(h, True, hp_ref)
    _logits(h, as_ref[...], ad_ref[...], es_ref, ed_ref, mh_ref)


def _tc_last_body(sp_ref, out_ref):
    s = sp_ref[0] + sp_ref[1]
    d = s[0:_N, 40:41]
    dsafe = jnp.where(d > 0.0, d, 1.0)
    logits = s[0:_N, 0:40] / dsafe
    m = jnp.max(logits, axis=1, keepdims=True)
    ex = jnp.exp(logits - m)
    lse = jnp.log(jnp.sum(ex, axis=1, keepdims=True)) + m
    out_ref[...] = logits - lse


def _vec_shapes():
    return [jax.ShapeDtypeStruct((_NPAD, 1), jnp.float32)] * 3


def _tc_split(h_fn):
    return pl.pallas_call(
        functools.partial(_tc_split_body, h_fn),
        out_shape=[jax.ShapeDtypeStruct((_NPAD, 80), jnp.float32)] * 2
        + _vec_shapes(),
    )


_tc_mid48 = pl.pallas_call(
    _tc_mid48_body,
    out_shape=[jax.ShapeDtypeStruct((_NPAD, 48), jnp.float32)] + _vec_shapes(),
)

_tc_last = pl.pallas_call(
    _tc_last_body,
    out_shape=jax.ShapeDtypeStruct((_N, 40), jnp.float32),
)


_SC_PARAMS = pltpu.CompilerParams(
    needs_layout_passes=False, use_tc_tiling_on_sc=False)


def _sc_scratch(fp):
    return [
        pltpu.VMEM((_NPAD,), jnp.float32),
        pltpu.VMEM((_NPAD,), jnp.float32),
        pltpu.VMEM((_NPAD,), jnp.float32),
        pltpu.VMEM((_CHUNK,), jnp.int32),
        pltpu.VMEM((_CHUNK,), jnp.int32),
        pltpu.VMEM((_CHUNK,), jnp.float32),
        pltpu.VMEM((_CHUNK, fp), jnp.float32),
        pltpu.VMEM_SHARED((_NPAD, fp), jnp.float32),
        pltpu.SemaphoreType.DMA,
    ]


def _zero_accumulator(rows, s_sh, tid, nvec):
    def _zrow(r, carry):
        for j in range(nvec):
            rows[r, pl.ds(j * 16, 16)] = jnp.zeros((16,), jnp.float32)
        return carry
    lax.fori_loop(0, _CHUNK, _zrow, 0)
    obase = tid * _RPT
    for kk in range(_RPT // _CHUNK):
        pltpu.sync_copy(rows, s_sh.at[pl.ds(obase + kk * _CHUNK, _CHUNK)])


def _edge_weights(src_v, dst_v, es_loc, ed_loc, mh_loc, w_v):
    for g in range(_CHUNK // 16):
        si = src_v[pl.ds(g * 16, 16)]
        di = dst_v[pl.ds(g * 16, 16)]
        e = plsc.load_gather(es_loc, [si]) + plsc.load_gather(ed_loc, [di])
        e = jnp.maximum(e, 0.2 * e) - plsc.load_gather(mh_loc, [di])
        w_v[pl.ds(g * 16, 16)] = jnp.exp(e)


def _scale_rows(rows, w_v, nvec):
    def _srow(r, carry):
        wspl = plsc.load_gather(w_v, [jnp.full((16,), r, jnp.int32)])
        for j in range(nvec):
            rows[r, pl.ds(j * 16, 16)] = rows[r, pl.ds(j * 16, 16)] * wspl
        return carry
    lax.fori_loop(0, _CHUNK, _srow, 0)


def _write_out(rows, s_sh, out_ref, tid):
    obase = tid * _RPT
    for kk in range(_RPT // _CHUNK):
        pltpu.sync_copy(s_sh.at[pl.ds(obase + kk * _CHUNK, _CHUNK)], rows)
        pltpu.sync_copy(rows, out_ref.at[pl.ds(obase + kk * _CHUNK, _CHUNK)])


def _sc_edge_featsplit():
    fp = 80
    nvec = fp // 16
    mesh = plsc.VectorSubcoreMesh(
        core_axis_name="c", subcore_axis_name="s",
        num_cores=_NC, num_subcores=_NS)

    @functools.partial(
        pl.kernel, mesh=mesh, compiler_params=_SC_PARAMS,
        out_type=jax.ShapeDtypeStruct((_NC, _NPAD, fp), jnp.float32),
        scratch_types=_sc_scratch(fp),
    )
    def k(src_hbm, dst_hbm, es_hbm, ed_hbm, mh_hbm, h0_hbm, h1_hbm, out_hbm,
          es_loc, ed_loc, mh_loc, src_v, dst_v, w_v, rows, s_sh, sem):
        cid = lax.axis_index("c")
        tid = lax.axis_index("s")

        _zero_accumulator(rows, s_sh, tid, nvec)
        pltpu.sync_copy(es_hbm, es_loc)
        pltpu.sync_copy(ed_hbm, ed_loc)
        pltpu.sync_copy(mh_hbm, mh_loc)
        plsc.subcore_barrier()

        ebase = tid * _EPT16

        def _chunk(c, carry):
            off = ebase + c * _CHUNK
            pltpu.sync_copy(src_hbm.at[pl.ds(off, _CHUNK)], src_v)
            pltpu.sync_copy(dst_hbm.at[pl.ds(off, _CHUNK)], dst_v)

            @pl.when(cid == 0)
            def _():
                pltpu.async_copy(h0_hbm.at[src_v], rows, sem).wait()

            @pl.when(cid == 1)
            def _():
                pltpu.async_copy(h1_hbm.at[src_v], rows, sem).wait()

            _edge_weights(src_v, dst_v, es_loc, ed_loc, mh_loc, w_v)
            _scale_rows(rows, w_v, nvec)
            pltpu.sync_copy(rows, s_sh.at[dst_v], add=True)
            return carry
        lax.fori_loop(0, _NCH16, _chunk, 0)

        plsc.subcore_barrier()
        _write_out(rows, s_sh, out_hbm.at[cid], tid)

    return k


def _sc_edge_edgesplit():
    fp = 48
    nvec = fp // 16
    mesh = plsc.VectorSubcoreMesh(
        core_axis_name="c", subcore_axis_name="s",
        num_cores=_NC, num_subcores=_NS)

    @functools.partial(
        pl.kernel, mesh=mesh, compiler_params=_SC_PARAMS,
        out_type=jax.ShapeDtypeStruct((_NC, _NPAD, fp), jnp.float32),
        scratch_types=_sc_scratch(fp),
    )
    def k(src_hbm, dst_hbm, es_hbm, ed_hbm, mh_hbm, h_hbm, out_hbm,
          es_loc, ed_loc, mh_loc, src_v, dst_v, w_v, rows, s_sh, sem):
        cid = lax.axis_index("c")
        sid = lax.axis_index("s")
        wid = sid * _NC + cid
        tid = sid

        _zero_accumulator(rows, s_sh, tid, nvec)
        pltpu.sync_copy(es_hbm, es_loc)
        pltpu.sync_copy(ed_hbm, ed_loc)
        pltpu.sync_copy(mh_hbm, mh_loc)
        plsc.subcore_barrier()

        ebase = wid * _EPT32

        def _chunk(c, carry):
            off = ebase + c * _CHUNK
            pltpu.sync_copy(src_hbm.at[pl.ds(off, _CHUNK)], src_v)
            pltpu.sync_copy(dst_hbm.at[pl.ds(off, _CHUNK)], dst_v)
            pltpu.async_copy(h_hbm.at[src_v], rows, sem).wait()
            _edge_weights(src_v, dst_v, es_loc, ed_loc, mh_loc, w_v)
            _scale_rows(rows, w_v, nvec)
            pltpu.sync_copy(rows, s_sh.at[dst_v], add=True)
            return carry
        lax.fori_loop(0, _NCH32, _chunk, 0)

        plsc.subcore_barrier()
        _write_out(rows, s_sh, out_hbm.at[cid], tid)

    return k


def _padded_edges(src, dst, epad):
    pad = jnp.full((epad - _E,), _SENT, jnp.int32)
    return jnp.concatenate([src, pad]), jnp.concatenate([dst, pad])


def kernel(x, edge_index, W1, a_src1, a_dst1, W2, a_src2, a_dst2,
           W3, a_src3, a_dst3):
    src = edge_index[0].astype(jnp.int32)
    dst = edge_index[1].astype(jnp.int32)
    src16, dst16 = _padded_edges(src, dst, _EPAD16)
    src32, dst32 = _padded_edges(src, dst, _EPAD32)

    sc_split = _sc_edge_featsplit()
    sc_part = _sc_edge_edgesplit()

    def _flat(a):
        return a.reshape(-1)

    h0, h1, es, ed, mh = _tc_split(_h_first)(
        x, W1, a_src1.reshape(1, -1), a_dst1.reshape(1, -1))
    s1 = sc_split(src16, dst16, _flat(es), _flat(ed), _flat(mh), h0, h1)
    h0, h1, es, ed, mh = _tc_split(_h_mid_split)(
        s1, W2, a_src2.reshape(1, -1), a_dst2.reshape(1, -1))
    s2 = sc_split(src16, dst16, _flat(es), _flat(ed), _flat(mh), h0, h1)
    hp, es, ed, mh = _tc_mid48(
        s2, W3, a_src3.reshape(1, -1), a_dst3.reshape(1, -1))
    s3 = sc_part(src32, dst32, _flat(es), _flat(ed), _flat(mh), hp)
    return _tc_last(s3)
